# baseline (speedup 1.0000x reference)
import jax
import jax.numpy as jnp
import numpy as np
from jax.experimental import pallas as pl
from jax.experimental.pallas import tpu as pltpu

_BP = 512
_T = 20
_NCK = 4

_BLKMASK = np.kron(np.eye(_T, dtype=np.float32), np.ones((3, 128), np.float32))


def _enc_kernel(pts_ref, pose_ref, goal_ref, blkmask_ref, W1_ref, b1_ref,
                W2_ref, b2_ref, W3a_ref, W3b_ref, b3_ref, W4_ref, b4_ref,
                Wg_ref, bg_ref, out_ref):
    pb = pl.program_id(1)
    T = _T
    BP = _BP

    x0 = pose_ref[0, :, 0:1]
    y0 = pose_ref[0, :, 1:2]
    th0 = pose_ref[0, :, 2:3]
    c = jnp.cos(th0)
    s = jnp.sin(th0)

    w1x = W1_ref[0:1, :]
    w1y = W1_ref[1:2, :]
    w1t = W1_ref[2:3, :]
    w1p = jnp.concatenate([c * w1x - s * w1y,
                           s * w1x + c * w1y,
                           w1t], axis=0)
    b1p = (b1_ref[:] - (x0 * c + y0 * s) * w1x
           + (x0 * s - y0 * c) * w1y - th0 * w1t)
    b1ph = jnp.tile(b1p, (1, T)).astype(jnp.bfloat16)

    wblk = (jnp.tile(w1p, (T, T)) * blkmask_ref[:]).astype(jnp.bfloat16)

    pts = pts_ref[0].astype(jnp.bfloat16)
    w23a = jnp.dot(W2_ref[:], W3a_ref[:],
                   preferred_element_type=jnp.float32).astype(jnp.bfloat16)
    r2a = jnp.dot(b2_ref[:].astype(jnp.bfloat16), W3a_ref[:],
                  preferred_element_type=jnp.float32)

    CK = BP // _NCK
    parts = []
    for ck in range(_NCK):
        h1p = jnp.dot(pts[ck * CK:(ck + 1) * CK], wblk,
                      preferred_element_type=jnp.float32).astype(jnp.bfloat16)
        h1p = jnp.maximum(h1p + b1ph, 0)

        h1 = jnp.concatenate(
            [h1p[:, t * 128:(t + 1) * 128] for t in range(T)], axis=0)
        h_nb = jnp.dot(h1, W2_ref[:],
                       preferred_element_type=jnp.float32).astype(jnp.bfloat16)

        pooled_nb = jnp.max(h_nb.reshape(T, CK, 256), axis=0)
        pooled = (pooled_nb.astype(jnp.float32) + b2_ref[:]).astype(jnp.bfloat16)

        a_nb = jnp.dot(h1, w23a,
                       preferred_element_type=jnp.float32).astype(jnp.bfloat16)
        bigc = (jnp.dot(pooled, W3b_ref[:], preferred_element_type=jnp.float32)
                + r2a + b3_ref[:]).astype(jnp.bfloat16)
        o1 = jnp.maximum(a_nb.reshape(T, CK, 256) + bigc[None, :, :], 0)
        o2_nb = jnp.dot(o1.reshape(T * CK, 256), W4_ref[:],
                        preferred_element_type=jnp.float32).astype(jnp.bfloat16)
        feat = jnp.max(o2_nb.reshape(T, CK, 256), axis=0)
        parts.append(jnp.sum(feat.astype(jnp.float32), axis=0, keepdims=True))
    part = sum(parts[1:], parts[0]) + BP * b4_ref[:]

    @pl.when(pb == 0)
    def _init():
        gx = goal_ref[0, :, 0:1] - x0
        gy = goal_ref[0, :, 1:2] - y0
        glx = gx * c + gy * s
        gly = gy * c - gx * s
        glth = goal_ref[0, :, 2:3] - th0
        out_ref[0] = (glx * Wg_ref[0:1, :] + gly * Wg_ref[1:2, :]
                      + glth * Wg_ref[2:3, :] + bg_ref[:])

    out_ref[0] += part


def kernel(goal, pose, map_polylines, W1, b1, W2, b2, W3, b3, W4, b4, Wg, bg):
    B, M, P, T, C = map_polylines.shape
    D = Wg.shape[1]
    BM = B * M
    BP = _BP
    nb = P // BP

    pts = map_polylines.reshape(BM, P, T * C)
    pose2 = pose.reshape(BM, 1, C)
    goal2 = goal.reshape(BM, 1, C)
    W2h = W2.astype(jnp.bfloat16)
    W3a, W3b = W3[:256].astype(jnp.bfloat16), W3[256:].astype(jnp.bfloat16)
    W4h = W4.astype(jnp.bfloat16)
    b1r, b2r = b1.reshape(1, -1), b2.reshape(1, -1)
    b3r, b4r = b3.reshape(1, -1), b4.reshape(1, -1)
    bgr = bg.reshape(1, -1)

    wspec = lambda shape: pl.BlockSpec(shape, lambda bm, pb: (0, 0))
    out = pl.pallas_call(
        _enc_kernel,
        grid=(BM, nb),
        in_specs=[
            pl.BlockSpec((1, BP, T * C), lambda bm, pb: (bm, pb, 0)),
            pl.BlockSpec((1, 1, C), lambda bm, pb: (bm, 0, 0)),
            pl.BlockSpec((1, 1, C), lambda bm, pb: (bm, 0, 0)),
            wspec(_BLKMASK.shape),
            wspec(W1.shape),
            wspec((1, b1.shape[0])),
            wspec(W2.shape),
            wspec((1, b2.shape[0])),
            wspec(W3a.shape),
            wspec(W3b.shape),
            wspec((1, b3.shape[0])),
            wspec(W4.shape),
            wspec((1, b4.shape[0])),
            wspec(Wg.shape),
            wspec((1, bg.shape[0])),
        ],
        out_specs=pl.BlockSpec((1, 1, D), lambda bm, pb: (bm, 0, 0)),
        out_shape=jax.ShapeDtypeStruct((BM, 1, D), jnp.float32),
        compiler_params=pltpu.CompilerParams(
            dimension_semantics=("parallel", "arbitrary")),
    )(pts, pose2, goal2, jnp.asarray(_BLKMASK), W1, b1r, W2h, b2r, W3a, W3b, b3r, W4h, b4r, Wg, bgr)
    return out.reshape(B, M, D)


# speedup vs baseline: 1.1001x; 1.1001x over previous
import jax
import jax.numpy as jnp
import numpy as np
from jax.experimental import pallas as pl
from jax.experimental.pallas import tpu as pltpu

_BP = 512
_T = 20
_NCK = 2
_NBM = 2

_BLKMASK = np.kron(np.eye(_T, dtype=np.float32), np.ones((3, 128), np.float32))


def _enc_kernel(pts_ref, pose_ref, goal_ref, blkmask_ref, W1_ref, b1_ref,
                W2_ref, b2_ref, W3a_ref, W3b_ref, b3_ref, W4_ref, b4_ref,
                Wg_ref, bg_ref, out_ref):
    T = _T
    BP = _BP

    for bm in range(_NBM):
        x0 = pose_ref[bm, :, 0:1]
        y0 = pose_ref[bm, :, 1:2]
        th0 = pose_ref[bm, :, 2:3]
        c = jnp.cos(th0)
        s = jnp.sin(th0)

        w1x = W1_ref[0:1, :]
        w1y = W1_ref[1:2, :]
        w1t = W1_ref[2:3, :]
        w1p = jnp.concatenate([c * w1x - s * w1y,
                               s * w1x + c * w1y,
                               w1t], axis=0)
        b1p = (b1_ref[:] - (x0 * c + y0 * s) * w1x
               + (x0 * s - y0 * c) * w1y - th0 * w1t)
        b1ph = jnp.tile(b1p, (1, T)).astype(jnp.bfloat16)

        wblk = (jnp.tile(w1p, (T, T)) * blkmask_ref[:]).astype(jnp.bfloat16)

        pts = pts_ref[bm].astype(jnp.bfloat16)
        w23a = jnp.dot(W2_ref[:], W3a_ref[:],
                       preferred_element_type=jnp.float32).astype(jnp.bfloat16)
        r2a = jnp.dot(b2_ref[:].astype(jnp.bfloat16), W3a_ref[:],
                      preferred_element_type=jnp.float32)

        CK = BP // _NCK
        parts = []
        for ck in range(_NCK):
            h1p = jnp.dot(pts[ck * CK:(ck + 1) * CK], wblk,
                          preferred_element_type=jnp.float32).astype(jnp.bfloat16)
            h1p = jnp.maximum(h1p + b1ph, 0)

            h1 = jnp.concatenate(
                [h1p[:, t * 128:(t + 1) * 128] for t in range(T)], axis=0)
            h_nb = jnp.dot(h1, W2_ref[:],
                           preferred_element_type=jnp.float32).astype(jnp.bfloat16)

            pooled_nb = jnp.max(h_nb.reshape(T, CK, 256), axis=0)
            pooled = (pooled_nb.astype(jnp.float32)
                      + b2_ref[:]).astype(jnp.bfloat16)

            a_nb = jnp.dot(h1, w23a,
                           preferred_element_type=jnp.float32).astype(jnp.bfloat16)
            bigc = (jnp.dot(pooled, W3b_ref[:],
                            preferred_element_type=jnp.float32)
                    + r2a + b3_ref[:]).astype(jnp.bfloat16)
            o1 = jnp.maximum(a_nb.reshape(T, CK, 256) + bigc[None, :, :], 0)
            o2_nb = jnp.dot(o1.reshape(T * CK, 256), W4_ref[:],
                            preferred_element_type=jnp.float32).astype(jnp.bfloat16)
            feat = jnp.max(o2_nb.reshape(T, CK, 256), axis=0)
            parts.append(jnp.sum(feat.astype(jnp.float32), axis=0,
                                 keepdims=True))
        part = sum(parts[1:], parts[0]) + BP * b4_ref[:]

        gx = goal_ref[bm, :, 0:1] - x0
        gy = goal_ref[bm, :, 1:2] - y0
        glx = gx * c + gy * s
        gly = gy * c - gx * s
        glth = goal_ref[bm, :, 2:3] - th0
        gf = (glx * Wg_ref[0:1, :] + gly * Wg_ref[1:2, :]
              + glth * Wg_ref[2:3, :] + bg_ref[:])
        out_ref[bm] = gf + part


def kernel(goal, pose, map_polylines, W1, b1, W2, b2, W3, b3, W4, b4, Wg, bg):
    B, M, P, T, C = map_polylines.shape
    D = Wg.shape[1]
    BM = B * M
    BP = _BP
    nb = P // BP

    pts = map_polylines.reshape(BM, P, T * C)
    pose2 = pose.reshape(BM, 1, C)
    goal2 = goal.reshape(BM, 1, C)
    W2h = W2.astype(jnp.bfloat16)
    W3a, W3b = W3[:256].astype(jnp.bfloat16), W3[256:].astype(jnp.bfloat16)
    W4h = W4.astype(jnp.bfloat16)
    b1r, b2r = b1.reshape(1, -1), b2.reshape(1, -1)
    b3r, b4r = b3.reshape(1, -1), b4.reshape(1, -1)
    bgr = bg.reshape(1, -1)

    wspec = lambda shape: pl.BlockSpec(shape, lambda i: (0, 0))
    out = pl.pallas_call(
        _enc_kernel,
        grid=(BM // _NBM,),
        in_specs=[
            pl.BlockSpec((_NBM, P, T * C), lambda i: (i, 0, 0)),
            pl.BlockSpec((_NBM, 1, C), lambda i: (i, 0, 0)),
            pl.BlockSpec((_NBM, 1, C), lambda i: (i, 0, 0)),
            wspec(_BLKMASK.shape),
            wspec(W1.shape),
            wspec((1, b1.shape[0])),
            wspec(W2.shape),
            wspec((1, b2.shape[0])),
            wspec(W3a.shape),
            wspec(W3b.shape),
            wspec((1, b3.shape[0])),
            wspec(W4.shape),
            wspec((1, b4.shape[0])),
            wspec(Wg.shape),
            wspec((1, bg.shape[0])),
        ],
        out_specs=pl.BlockSpec((_NBM, 1, D), lambda i: (i, 0, 0)),
        out_shape=jax.ShapeDtypeStruct((BM, 1, D), jnp.float32),
        compiler_params=pltpu.CompilerParams(
            dimension_semantics=("parallel",)),
    )(pts, pose2, goal2, jnp.asarray(_BLKMASK), W1, b1r, W2h, b2r, W3a, W3b, b3r, W4h, b4r, Wg, bgr)
    return out.reshape(B, M, D)


# speedup vs baseline: 1.1178x; 1.0162x over previous
import jax
import jax.numpy as jnp
import numpy as np
from jax.experimental import pallas as pl
from jax.experimental.pallas import tpu as pltpu

_BP = 512
_T = 20
_NCK = 1
_NBM = 2

_BLKMASK = np.kron(np.eye(_T, dtype=np.float32), np.ones((3, 128), np.float32))


def _enc_kernel(pts_ref, pose_ref, goal_ref, blkmask_ref, W1_ref, b1_ref,
                W2_ref, b2_ref, W3a_ref, W3b_ref, b3_ref, W4_ref, b4_ref,
                Wg_ref, bg_ref, out_ref):
    T = _T
    BP = _BP

    for bm in range(_NBM):
        x0 = pose_ref[bm, :, 0:1]
        y0 = pose_ref[bm, :, 1:2]
        th0 = pose_ref[bm, :, 2:3]
        c = jnp.cos(th0)
        s = jnp.sin(th0)

        w1x = W1_ref[0:1, :]
        w1y = W1_ref[1:2, :]
        w1t = W1_ref[2:3, :]
        w1p = jnp.concatenate([c * w1x - s * w1y,
                               s * w1x + c * w1y,
                               w1t], axis=0)
        b1p = (b1_ref[:] - (x0 * c + y0 * s) * w1x
               + (x0 * s - y0 * c) * w1y - th0 * w1t)
        b1ph = jnp.tile(b1p, (1, T)).astype(jnp.bfloat16)

        wblk = (jnp.tile(w1p, (T, T)) * blkmask_ref[:]).astype(jnp.bfloat16)

        pts = pts_ref[bm].astype(jnp.bfloat16)
        w23a = jnp.dot(W2_ref[:], W3a_ref[:],
                       preferred_element_type=jnp.float32).astype(jnp.bfloat16)
        r2a = jnp.dot(b2_ref[:].astype(jnp.bfloat16), W3a_ref[:],
                      preferred_element_type=jnp.float32)

        CK = BP // _NCK
        parts = []
        for ck in range(_NCK):
            h1p = jnp.dot(pts[ck * CK:(ck + 1) * CK], wblk,
                          preferred_element_type=jnp.float32).astype(jnp.bfloat16)
            h1p = jnp.maximum(h1p + b1ph, 0)

            h1 = jnp.concatenate(
                [h1p[:, t * 128:(t + 1) * 128] for t in range(T)], axis=0)
            h_nb = jnp.dot(h1, W2_ref[:],
                           preferred_element_type=jnp.float32).astype(jnp.bfloat16)

            pooled_nb = jnp.max(h_nb.reshape(T, CK, 256), axis=0)
            pooled = (pooled_nb.astype(jnp.float32)
                      + b2_ref[:]).astype(jnp.bfloat16)

            a_nb = jnp.dot(h1, w23a,
                           preferred_element_type=jnp.float32).astype(jnp.bfloat16)
            bigc = (jnp.dot(pooled, W3b_ref[:],
                            preferred_element_type=jnp.float32)
                    + r2a + b3_ref[:]).astype(jnp.bfloat16)
            o1 = jnp.maximum(a_nb.reshape(T, CK, 256) + bigc[None, :, :], 0)
            o2_nb = jnp.dot(o1.reshape(T * CK, 256), W4_ref[:],
                            preferred_element_type=jnp.float32).astype(jnp.bfloat16)
            feat = jnp.max(o2_nb.reshape(T, CK, 256), axis=0)
            parts.append(jnp.sum(feat.astype(jnp.float32), axis=0,
                                 keepdims=True))
        part = sum(parts[1:], parts[0]) + BP * b4_ref[:]

        gx = goal_ref[bm, :, 0:1] - x0
        gy = goal_ref[bm, :, 1:2] - y0
        glx = gx * c + gy * s
        gly = gy * c - gx * s
        glth = goal_ref[bm, :, 2:3] - th0
        gf = (glx * Wg_ref[0:1, :] + gly * Wg_ref[1:2, :]
              + glth * Wg_ref[2:3, :] + bg_ref[:])
        out_ref[bm] = gf + part


def kernel(goal, pose, map_polylines, W1, b1, W2, b2, W3, b3, W4, b4, Wg, bg):
    B, M, P, T, C = map_polylines.shape
    D = Wg.shape[1]
    BM = B * M
    BP = _BP
    nb = P // BP

    pts = map_polylines.reshape(BM, P, T * C)
    pose2 = pose.reshape(BM, 1, C)
    goal2 = goal.reshape(BM, 1, C)
    W2h = W2.astype(jnp.bfloat16)
    W3a, W3b = W3[:256].astype(jnp.bfloat16), W3[256:].astype(jnp.bfloat16)
    W4h = W4.astype(jnp.bfloat16)
    b1r, b2r = b1.reshape(1, -1), b2.reshape(1, -1)
    b3r, b4r = b3.reshape(1, -1), b4.reshape(1, -1)
    bgr = bg.reshape(1, -1)

    wspec = lambda shape: pl.BlockSpec(shape, lambda i: (0, 0))
    out = pl.pallas_call(
        _enc_kernel,
        grid=(BM // _NBM,),
        in_specs=[
            pl.BlockSpec((_NBM, P, T * C), lambda i: (i, 0, 0)),
            pl.BlockSpec((_NBM, 1, C), lambda i: (i, 0, 0)),
            pl.BlockSpec((_NBM, 1, C), lambda i: (i, 0, 0)),
            wspec(_BLKMASK.shape),
            wspec(W1.shape),
            wspec((1, b1.shape[0])),
            wspec(W2.shape),
            wspec((1, b2.shape[0])),
            wspec(W3a.shape),
            wspec(W3b.shape),
            wspec((1, b3.shape[0])),
            wspec(W4.shape),
            wspec((1, b4.shape[0])),
            wspec(Wg.shape),
            wspec((1, bg.shape[0])),
        ],
        out_specs=pl.BlockSpec((_NBM, 1, D), lambda i: (i, 0, 0)),
        out_shape=jax.ShapeDtypeStruct((BM, 1, D), jnp.float32),
        compiler_params=pltpu.CompilerParams(
            dimension_semantics=("parallel",)),
    )(pts, pose2, goal2, jnp.asarray(_BLKMASK), W1, b1r, W2h, b2r, W3a, W3b, b3r, W4h, b4r, Wg, bgr)
    return out.reshape(B, M, D)


# speedup vs baseline: 1.1373x; 1.0174x over previous
import jax
import jax.numpy as jnp
import numpy as np
from jax.experimental import pallas as pl
from jax.experimental.pallas import tpu as pltpu

_BP = 512
_T = 20
_NCK = 1
_NBM = 4

_BLKMASK = np.kron(np.eye(_T, dtype=np.float32), np.ones((3, 128), np.float32))


def _enc_kernel(pts_ref, pose_ref, goal_ref, blkmask_ref, W1_ref, b1_ref,
                W2_ref, b2_ref, W3a_ref, W3b_ref, b3_ref, W4_ref, b4_ref,
                Wg_ref, bg_ref, out_ref):
    T = _T
    BP = _BP

    for bm in range(_NBM):
        x0 = pose_ref[bm, :, 0:1]
        y0 = pose_ref[bm, :, 1:2]
        th0 = pose_ref[bm, :, 2:3]
        c = jnp.cos(th0)
        s = jnp.sin(th0)

        w1x = W1_ref[0:1, :]
        w1y = W1_ref[1:2, :]
        w1t = W1_ref[2:3, :]
        w1p = jnp.concatenate([c * w1x - s * w1y,
                               s * w1x + c * w1y,
                               w1t], axis=0)
        b1p = (b1_ref[:] - (x0 * c + y0 * s) * w1x
               + (x0 * s - y0 * c) * w1y - th0 * w1t)
        b1ph = jnp.tile(b1p, (1, T)).astype(jnp.bfloat16)

        wblk = (jnp.tile(w1p, (T, T)) * blkmask_ref[:]).astype(jnp.bfloat16)

        pts = pts_ref[bm].astype(jnp.bfloat16)
        w23a = jnp.dot(W2_ref[:], W3a_ref[:],
                       preferred_element_type=jnp.float32).astype(jnp.bfloat16)
        r2a = jnp.dot(b2_ref[:].astype(jnp.bfloat16), W3a_ref[:],
                      preferred_element_type=jnp.float32)

        CK = BP // _NCK
        parts = []
        for ck in range(_NCK):
            h1p = jnp.dot(pts[ck * CK:(ck + 1) * CK], wblk,
                          preferred_element_type=jnp.float32).astype(jnp.bfloat16)
            h1p = jnp.maximum(h1p + b1ph, 0)

            h1 = jnp.concatenate(
                [h1p[:, t * 128:(t + 1) * 128] for t in range(T)], axis=0)
            h_nb = jnp.dot(h1, W2_ref[:],
                           preferred_element_type=jnp.float32).astype(jnp.bfloat16)

            pooled_nb = jnp.max(h_nb.reshape(T, CK, 256), axis=0)
            pooled = (pooled_nb.astype(jnp.float32)
                      + b2_ref[:]).astype(jnp.bfloat16)

            a_nb = jnp.dot(h1, w23a,
                           preferred_element_type=jnp.float32).astype(jnp.bfloat16)
            bigc = (jnp.dot(pooled, W3b_ref[:],
                            preferred_element_type=jnp.float32)
                    + r2a + b3_ref[:]).astype(jnp.bfloat16)
            o1 = jnp.maximum(a_nb.reshape(T, CK, 256) + bigc[None, :, :], 0)
            o2_nb = jnp.dot(o1.reshape(T * CK, 256), W4_ref[:],
                            preferred_element_type=jnp.float32).astype(jnp.bfloat16)
            feat = jnp.max(o2_nb.reshape(T, CK, 256), axis=0)
            parts.append(jnp.sum(feat.astype(jnp.float32), axis=0,
                                 keepdims=True))
        part = sum(parts[1:], parts[0]) + BP * b4_ref[:]

        gx = goal_ref[bm, :, 0:1] - x0
        gy = goal_ref[bm, :, 1:2] - y0
        glx = gx * c + gy * s
        gly = gy * c - gx * s
        glth = goal_ref[bm, :, 2:3] - th0
        gf = (glx * Wg_ref[0:1, :] + gly * Wg_ref[1:2, :]
              + glth * Wg_ref[2:3, :] + bg_ref[:])
        out_ref[bm] = gf + part


def kernel(goal, pose, map_polylines, W1, b1, W2, b2, W3, b3, W4, b4, Wg, bg):
    B, M, P, T, C = map_polylines.shape
    D = Wg.shape[1]
    BM = B * M
    BP = _BP
    nb = P // BP

    pts = map_polylines.reshape(BM, P, T * C)
    pose2 = pose.reshape(BM, 1, C)
    goal2 = goal.reshape(BM, 1, C)
    W2h = W2.astype(jnp.bfloat16)
    W3a, W3b = W3[:256].astype(jnp.bfloat16), W3[256:].astype(jnp.bfloat16)
    W4h = W4.astype(jnp.bfloat16)
    b1r, b2r = b1.reshape(1, -1), b2.reshape(1, -1)
    b3r, b4r = b3.reshape(1, -1), b4.reshape(1, -1)
    bgr = bg.reshape(1, -1)

    wspec = lambda shape: pl.BlockSpec(shape, lambda i: (0, 0))
    out = pl.pallas_call(
        _enc_kernel,
        grid=(BM // _NBM,),
        in_specs=[
            pl.BlockSpec((_NBM, P, T * C), lambda i: (i, 0, 0)),
            pl.BlockSpec((_NBM, 1, C), lambda i: (i, 0, 0)),
            pl.BlockSpec((_NBM, 1, C), lambda i: (i, 0, 0)),
            wspec(_BLKMASK.shape),
            wspec(W1.shape),
            wspec((1, b1.shape[0])),
            wspec(W2.shape),
            wspec((1, b2.shape[0])),
            wspec(W3a.shape),
            wspec(W3b.shape),
            wspec((1, b3.shape[0])),
            wspec(W4.shape),
            wspec((1, b4.shape[0])),
            wspec(Wg.shape),
            wspec((1, bg.shape[0])),
        ],
        out_specs=pl.BlockSpec((_NBM, 1, D), lambda i: (i, 0, 0)),
        out_shape=jax.ShapeDtypeStruct((BM, 1, D), jnp.float32),
        compiler_params=pltpu.CompilerParams(
            dimension_semantics=("parallel",)),
    )(pts, pose2, goal2, jnp.asarray(_BLKMASK), W1, b1r, W2h, b2r, W3a, W3b, b3r, W4h, b4r, Wg, bgr)
    return out.reshape(B, M, D)
